# Initial kernel scaffold; baseline (speedup 1.0000x reference)
import jax
import jax.numpy as jnp
from jax.experimental import pallas as pl


def kernel(obs, action_embeddings, parent_embeddings, sibling_embeddings):
    raise NotImplementedError("write your pallas kernel here")


# speedup vs baseline: 3.4365x; 3.4365x over previous
import functools

import jax
import jax.numpy as jnp
from jax import lax
from jax.experimental import pallas as pl
from jax.experimental.pallas import tpu as pltpu
from jax.experimental.pallas import tpu_sc as plsc

B = 4096
T = 50
EMB = 128
N_EXT = 5
OUT = 3 * EMB + N_EXT
R = B * T

NC = 2
NS = 16
NW = NC * NS
ROWS_W = R // NW
C = 200
NB = C // T
SUB = 100
NSUB = C // SUB
NCHUNK = ROWS_W // C


def _sc_kernel_body(idxa_h, idxp_h, idxs_h, ext_h, ta_h, tp_h, ts_h, out_h,
                    idxa, idxp, idxs, rows0, rows1, rows2, exts, extd,
                    sem_a, sem_b, sem_c):
  wid = lax.axis_index("s") * NC + lax.axis_index("c")
  row0 = wid * ROWS_W
  iota = lax.iota(jnp.int32, 16)

  nsub_w = ROWS_W // SUB
  pltpu.sync_copy(idxa_h.at[pl.ds(wid * nsub_w, nsub_w)], idxa)
  pltpu.sync_copy(idxp_h.at[pl.ds(wid * nsub_w, nsub_w)], idxp)
  pltpu.sync_copy(idxs_h.at[pl.ds(wid * nsub_w, nsub_w)], idxs)

  def chunk_body(ci, carry):
    base = row0 + ci * C

    pltpu.sync_copy(ext_h.at[pl.ds(base * N_EXT, C * N_EXT)], exts)

    copies = []
    for k in range(NSUB):
      copies.append(pltpu.async_copy(
          ta_h.at[idxa.at[ci * NSUB + k]], rows0.at[pl.ds(k * SUB, SUB)],
          sem_a))
    for k in range(NSUB):
      copies.append(pltpu.async_copy(
          tp_h.at[idxp.at[ci * NSUB + k]], rows1.at[pl.ds(k * SUB, SUB)],
          sem_b))
    for k in range(NSUB):
      copies.append(pltpu.async_copy(
          ts_h.at[idxs.at[ci * NSUB + k]], rows2.at[pl.ds(k * SUB, SUB)],
          sem_c))

    for bb in range(NB):
      for j in range(N_EXT):
        src = bb * (N_EXT * T) + j * T
        for t0 in (0, 16, 32, 34):
          v = exts[pl.ds(src + t0, 16)]
          r = jnp.int32(bb * T + t0) + iota
          plsc.store_scatter(extd, [r, jnp.full((16,), j, jnp.int32)], v)
    pltpu.sync_copy(extd, out_h.at[pl.ds(base, C), pl.ds(3 * EMB, N_EXT)])

    for k in range(NSUB):
      copies[k].wait()
    pltpu.sync_copy(rows0, out_h.at[pl.ds(base, C), pl.ds(0, EMB)])
    for k in range(NSUB):
      copies[NSUB + k].wait()
    pltpu.sync_copy(rows1, out_h.at[pl.ds(base, C), pl.ds(EMB, EMB)])
    for k in range(NSUB):
      copies[2 * NSUB + k].wait()
    pltpu.sync_copy(rows2, out_h.at[pl.ds(base, C), pl.ds(2 * EMB, EMB)])
    return carry

  lax.fori_loop(0, NCHUNK, chunk_body, 0)


@jax.jit
def _run(idxa, idxp, idxs, ext, ta, tp, ts):
  mesh = plsc.VectorSubcoreMesh(core_axis_name="c", subcore_axis_name="s")
  f = pl.kernel(
      _sc_kernel_body,
      out_type=jax.ShapeDtypeStruct((R, OUT), jnp.float32),
      mesh=mesh,
      compiler_params=pltpu.CompilerParams(needs_layout_passes=False),
      scratch_types=[
          pltpu.VMEM((ROWS_W // SUB, SUB), jnp.int32),
          pltpu.VMEM((ROWS_W // SUB, SUB), jnp.int32),
          pltpu.VMEM((ROWS_W // SUB, SUB), jnp.int32),
          pltpu.VMEM((C, EMB), jnp.float32),
          pltpu.VMEM((C, EMB), jnp.float32),
          pltpu.VMEM((C, EMB), jnp.float32),
          pltpu.VMEM((C * N_EXT,), jnp.float32),
          pltpu.VMEM((C, N_EXT), jnp.float32),
          pltpu.SemaphoreType.DMA,
          pltpu.SemaphoreType.DMA,
          pltpu.SemaphoreType.DMA,
      ],
  )
  return f(idxa, idxp, idxs, ext, ta, tp, ts)


def kernel(obs, action_embeddings, parent_embeddings, sibling_embeddings):
  idxa = obs[:, 0, :].astype(jnp.int32).reshape(R // SUB, SUB)
  idxp = obs[:, 1, :].astype(jnp.int32).reshape(R // SUB, SUB)
  idxs = obs[:, 2, :].astype(jnp.int32).reshape(R // SUB, SUB)
  ext = obs[:, 3:, :].reshape(B * N_EXT * T)
  out = _run(idxa, idxp, idxs, ext, action_embeddings, parent_embeddings,
             sibling_embeddings)
  return out.reshape(B, T, OUT)
